# baseline (speedup 1.0000x reference)
import functools

import jax
import jax.numpy as jnp
from jax import lax
from jax.experimental import pallas as pl
from jax.experimental.pallas import tpu as pltpu
from jax.experimental.pallas import tpu_sc as plsc

_TM = 512
_WIN = 2816


def _round_bf16(x):
    xi = x.view(jnp.uint32)
    r = (xi + jnp.uint32(0x7FFF) + ((xi >> jnp.uint32(16)) & jnp.uint32(1)))
    return (r & jnp.uint32(0xFFFF0000)).view(jnp.float32)


def _argmin_body(cb_ref, z_ref, out_ref, cn_ref, iota_ref):
    v = cb_ref.shape[0]

    @pl.when(pl.program_id(0) == 0)
    def _init():
        cb0 = cb_ref[...]
        cn_ref[...] = jnp.sum(cb0 * cb0, axis=1)[None, :]
        iota_ref[...] = lax.broadcasted_iota(
            jnp.int32, (1, v), 1).astype(jnp.float32)

    z = z_ref[...]
    mm2 = lax.dot_general((z * -2.0).astype(jnp.bfloat16),
                          cb_ref[...].astype(jnp.bfloat16),
                          (((1,), (1,)), ((), ())),
                          preferred_element_type=jnp.float32)
    zn = jnp.sum(z * z, axis=1, keepdims=True)
    d = (zn + cn_ref[...]) + mm2
    acc_v = None
    for lo in range(0, v, _WIN):
        hi = min(lo + _WIN, v)
        dw = d[:, lo:hi]
        m = jnp.min(dw, axis=1)
        idx = jnp.min(jnp.where(dw == m[:, None],
                                iota_ref[:, lo:hi], float(v)), axis=1)
        if acc_v is None:
            acc_v, acc_i = _round_bf16(m), idx
        else:
            won = (m < acc_v) | ((m == acc_v) & (idx < acc_i))
            acc_i = jnp.where(won, idx, acc_i)
            acc_v = jnp.where(won, _round_bf16(m), acc_v)
    out_ref[...] = acc_i.astype(jnp.int32)[None, None, :]


def _tc_argmin(z, codebook):
    n, d_model = z.shape
    v = codebook.shape[0]
    grid = n // _TM
    idx3 = pl.pallas_call(
        _argmin_body,
        grid=(grid,),
        in_specs=[
            pl.BlockSpec((v, d_model), lambda i: (0, 0)),
            pl.BlockSpec((_TM, d_model), lambda i: (i, 0)),
        ],
        out_specs=pl.BlockSpec((1, 1, _TM), lambda i: (i, 0, 0)),
        out_shape=jax.ShapeDtypeStruct((grid, 1, _TM), jnp.int32),
        scratch_shapes=[
            pltpu.VMEM((1, v), jnp.float32),
            pltpu.VMEM((1, v), jnp.float32),
        ],
        compiler_params=pltpu.CompilerParams(
            dimension_semantics=("arbitrary",)),
    )(codebook, z)
    return idx3.reshape(n)


def _make_sc_gather(v, d_model, n):
    info = plsc.get_sparse_core_info()
    nw = info.num_cores * info.num_subcores
    b_per_w = n // nw
    chunk = 128
    n_chunks = b_per_w // chunk
    mesh = plsc.VectorSubcoreMesh(core_axis_name="c", subcore_axis_name="s")

    @functools.partial(
        pl.kernel,
        mesh=mesh,
        out_type=jax.ShapeDtypeStruct((n, d_model), jnp.float32),
        scratch_types=[
            pltpu.VMEM((chunk,), jnp.int32),
            pltpu.VMEM((chunk, d_model), jnp.float32),
            pltpu.SemaphoreType.DMA,
        ],
    )
    def gather(table_hbm, idx_hbm, out_hbm, idx_v, rows_v, sem):
        wid = lax.axis_index("s") * info.num_cores + lax.axis_index("c")
        for ch in range(n_chunks):
            base = wid * b_per_w + ch * chunk
            pltpu.sync_copy(idx_hbm.at[pl.ds(base, chunk)], idx_v)
            pltpu.async_copy(table_hbm.at[idx_v], rows_v, sem).wait()
            pltpu.sync_copy(rows_v, out_hbm.at[pl.ds(base, chunk)])

    return gather


def kernel(ze, codebook):
    b, s, d_model = ze.shape
    v = codebook.shape[0]
    n = b * s
    z = ze.reshape(n, d_model)
    idx = _tc_argmin(z, codebook)
    zq = _make_sc_gather(v, d_model, n)(codebook, idx)
    return zq.reshape(ze.shape)


# speedup vs baseline: 1.1258x; 1.1258x over previous
import functools

import jax
import jax.numpy as jnp
from jax import lax
from jax.experimental import pallas as pl
from jax.experimental.pallas import tpu as pltpu
from jax.experimental.pallas import tpu_sc as plsc

_TM = 1024
_WIN = 2816


def _round_bf16(x):
    xi = x.view(jnp.uint32)
    r = (xi + jnp.uint32(0x7FFF) + ((xi >> jnp.uint32(16)) & jnp.uint32(1)))
    return (r & jnp.uint32(0xFFFF0000)).view(jnp.float32)


def _argmin_body(cb_ref, z_ref, out_ref, cn_ref, iota_ref):
    v = cb_ref.shape[0]

    @pl.when(pl.program_id(0) == 0)
    def _init():
        cb0 = cb_ref[...]
        cn_ref[...] = jnp.sum(cb0 * cb0, axis=1)[None, :]
        iota_ref[...] = lax.broadcasted_iota(
            jnp.int32, (1, v), 1).astype(jnp.float32)

    z = z_ref[...]
    mm = lax.dot_general(z.astype(jnp.bfloat16), cb_ref[...].astype(jnp.bfloat16),
                         (((1,), (1,)), ((), ())),
                         preferred_element_type=jnp.float32)
    zn = jnp.sum(z * z, axis=1, keepdims=True)
    d = (zn + cn_ref[...]) - 2.0 * mm
    acc_v = None
    for lo in range(0, v, _WIN):
        hi = min(lo + _WIN, v)
        dw = d[:, lo:hi]
        m = jnp.min(dw, axis=1)
        idx = jnp.min(jnp.where(dw == m[:, None],
                                iota_ref[:, lo:hi], float(v)), axis=1)
        if acc_v is None:
            acc_v, acc_i = _round_bf16(m), idx
        else:
            won = (m < acc_v) | ((m == acc_v) & (idx < acc_i))
            acc_i = jnp.where(won, idx, acc_i)
            acc_v = jnp.where(won, _round_bf16(m), acc_v)
    out_ref[...] = acc_i.astype(jnp.int32)[None, None, :]


def _tc_argmin(z, codebook):
    n, d_model = z.shape
    v = codebook.shape[0]
    grid = n // _TM
    idx3 = pl.pallas_call(
        _argmin_body,
        grid=(grid,),
        in_specs=[
            pl.BlockSpec((v, d_model), lambda i: (0, 0)),
            pl.BlockSpec((_TM, d_model), lambda i: (i, 0)),
        ],
        out_specs=pl.BlockSpec((1, 1, _TM), lambda i: (i, 0, 0)),
        out_shape=jax.ShapeDtypeStruct((grid, 1, _TM), jnp.int32),
        scratch_shapes=[
            pltpu.VMEM((1, v), jnp.float32),
            pltpu.VMEM((1, v), jnp.float32),
        ],
        compiler_params=pltpu.CompilerParams(
            dimension_semantics=("arbitrary",)),
    )(codebook, z)
    return idx3.reshape(n)


def _make_sc_gather(v, d_model, n):
    info = plsc.get_sparse_core_info()
    nw = info.num_cores * info.num_subcores
    b_per_w = n // nw
    chunk = 128
    n_chunks = b_per_w // chunk
    mesh = plsc.VectorSubcoreMesh(core_axis_name="c", subcore_axis_name="s")

    @functools.partial(
        pl.kernel,
        mesh=mesh,
        out_type=jax.ShapeDtypeStruct((n, d_model), jnp.float32),
        scratch_types=[
            pltpu.VMEM((chunk,), jnp.int32),
            pltpu.VMEM((chunk, d_model), jnp.float32),
            pltpu.SemaphoreType.DMA,
        ],
    )
    def gather(table_hbm, idx_hbm, out_hbm, idx_v, rows_v, sem):
        wid = lax.axis_index("s") * info.num_cores + lax.axis_index("c")
        for ch in range(n_chunks):
            base = wid * b_per_w + ch * chunk
            pltpu.sync_copy(idx_hbm.at[pl.ds(base, chunk)], idx_v)
            pltpu.async_copy(table_hbm.at[idx_v], rows_v, sem).wait()
            pltpu.sync_copy(rows_v, out_hbm.at[pl.ds(base, chunk)])

    return gather


def kernel(ze, codebook):
    b, s, d_model = ze.shape
    v = codebook.shape[0]
    n = b * s
    z = ze.reshape(n, d_model)
    idx = _tc_argmin(z, codebook)
    zq = _make_sc_gather(v, d_model, n)(codebook, idx)
    return zq.reshape(ze.shape)
